# baseline (speedup 1.0000x reference)
import functools

import jax
import jax.numpy as jnp
from jax import lax
from jax.experimental import pallas as pl
from jax.experimental.pallas import tpu as pltpu
from jax.experimental.pallas import tpu_sc as plsc

N_CORES = 2
N_SUBCORES = 16
NW = N_CORES * N_SUBCORES
CHUNK = 128
BLK = 512


def _pad_up(x, m):
    return (x + m - 1) // m * m


def _sc_mesh():
    return plsc.VectorSubcoreMesh(core_axis_name="c", subcore_axis_name="s")


_SC_PARAMS = pltpu.CompilerParams(use_tc_tiling_on_sc=False)


def _sc_hist(row_idx, n_pad, cpt):
    rows_per_tile = n_pad // N_SUBCORES
    slabs = rows_per_tile // CHUNK

    @functools.partial(
        pl.kernel,
        out_type=jax.ShapeDtypeStruct((N_CORES, n_pad, 16), jnp.float32),
        mesh=_sc_mesh(),
        scratch_types=[
            pltpu.VMEM((cpt, CHUNK), jnp.int32),
            pltpu.VMEM((CHUNK, 16), jnp.float32),
            pltpu.VMEM((CHUNK, 16), jnp.float32),
            pltpu.VMEM_SHARED((n_pad, 16), jnp.float32),
            pltpu.SemaphoreType.DMA,
        ],
        compiler_params=_SC_PARAMS,
    )
    def hist_kernel(row_hbm, out_hbm, idx_v, ones_v, zeros_v, hist_s, sem):
        c = lax.axis_index("c")
        s = lax.axis_index("s")
        w = c * N_SUBCORES + s
        pltpu.async_copy(row_hbm.at[w], idx_v, sem).wait()

        @pl.loop(0, CHUNK)
        def _(i):
            ones_v[i, :] = jnp.full((16,), 1.0, jnp.float32)
            zeros_v[i, :] = jnp.zeros((16,), jnp.float32)

        @pl.loop(0, slabs)
        def _(k):
            pltpu.sync_copy(
                zeros_v, hist_s.at[pl.ds(s * rows_per_tile + k * CHUNK, CHUNK)]
            )

        plsc.subcore_barrier()

        @pl.loop(0, cpt)
        def _(j):
            pltpu.sync_copy(ones_v, hist_s.at[idx_v.at[j]], add=True)

        plsc.subcore_barrier()
        pltpu.sync_copy(
            hist_s.at[pl.ds(s * rows_per_tile, rows_per_tile)],
            out_hbm.at[c].at[pl.ds(s * rows_per_tile, rows_per_tile)],
        )

    return hist_kernel(row_idx)


def _sc_scatter(col_idx, row_idx, z, n_pad, cpt, d):
    rows_per_tile = n_pad // N_SUBCORES
    slabs = rows_per_tile // CHUNK

    @functools.partial(
        pl.kernel,
        out_type=jax.ShapeDtypeStruct((N_CORES, n_pad, d), jnp.float32),
        mesh=_sc_mesh(),
        scratch_types=[
            pltpu.VMEM((cpt, CHUNK), jnp.int32),
            pltpu.VMEM((cpt, CHUNK), jnp.int32),
            pltpu.VMEM((CHUNK, d), jnp.float32),
            pltpu.VMEM((CHUNK, d), jnp.float32),
            pltpu.VMEM((CHUNK, d), jnp.float32),
            pltpu.VMEM_SHARED((n_pad, d), jnp.float32),
            pltpu.SemaphoreType.DMA,
            pltpu.SemaphoreType.DMA,
            pltpu.SemaphoreType.DMA,
        ],
        compiler_params=_SC_PARAMS,
    )
    def scat_kernel(col_hbm, row_hbm, z_hbm, out_hbm, cidx_v, ridx_v, buf_a,
                    buf_b, zeros_v, acc_s, sem_a, sem_b, sem):
        c = lax.axis_index("c")
        s = lax.axis_index("s")
        w = c * N_SUBCORES + s
        pltpu.async_copy(col_hbm.at[w], cidx_v, sem).wait()
        pltpu.async_copy(row_hbm.at[w], ridx_v, sem).wait()

        @pl.loop(0, CHUNK)
        def _(i):
            @pl.loop(0, d // 16)
            def _(t):
                zeros_v[i, pl.ds(t * 16, 16)] = jnp.zeros((16,), jnp.float32)

        @pl.loop(0, slabs)
        def _(k):
            pltpu.sync_copy(
                zeros_v, acc_s.at[pl.ds(s * rows_per_tile + k * CHUNK, CHUNK)]
            )

        plsc.subcore_barrier()

        pltpu.async_copy(z_hbm.at[cidx_v.at[0]], buf_a, sem_a)

        @pl.loop(0, cpt, step=2)
        def _(j):
            pltpu.async_copy(z_hbm.at[cidx_v.at[j + 1]], buf_b, sem_b)
            pltpu.make_async_copy(z_hbm.at[cidx_v.at[j]], buf_a, sem_a).wait()
            pltpu.sync_copy(buf_a, acc_s.at[ridx_v.at[j]], add=True)

            @pl.when(j + 2 < cpt)
            def _():
                pltpu.async_copy(z_hbm.at[cidx_v.at[j + 2]], buf_a, sem_a)

            pltpu.make_async_copy(z_hbm.at[cidx_v.at[j + 1]], buf_b, sem_b).wait()
            pltpu.sync_copy(buf_b, acc_s.at[ridx_v.at[j + 1]], add=True)

        plsc.subcore_barrier()
        pltpu.sync_copy(
            acc_s.at[pl.ds(s * rows_per_tile, rows_per_tile)],
            out_hbm.at[c].at[pl.ds(s * rows_per_tile, rows_per_tile)],
        )

    return scat_kernel(col_idx, row_idx, z)


def _tc_matmul(x, w):
    n, k = x.shape
    m = w.shape[1]
    grid = (n + BLK - 1) // BLK

    def body(x_ref, w_ref, o_ref):
        o_ref[...] = jnp.dot(x_ref[...], w_ref[...],
                             preferred_element_type=jnp.float32)

    return pl.pallas_call(
        body,
        grid=(grid,),
        in_specs=[
            pl.BlockSpec((BLK, k), lambda i: (i, 0)),
            pl.BlockSpec((k, m), lambda i: (0, 0)),
        ],
        out_specs=pl.BlockSpec((BLK, m), lambda i: (i, 0)),
        out_shape=jax.ShapeDtypeStruct((n, m), jnp.float32),
    )(x, w)


def _tc_scale(hist, y):
    n, m = y.shape
    mh = m // 2
    grid = (n + BLK - 1) // BLK

    def body(h_ref, y_ref, d_ref, z_ref):
        deg = h_ref[0, :, 0:1] + h_ref[1, :, 0:1] + 1.0
        dv = lax.rsqrt(deg)
        d_ref[...] = dv
        z = y_ref[...] * dv
        z_ref[0, :, :] = z[:, :mh]
        z_ref[1, :, :] = z[:, mh:]

    return pl.pallas_call(
        body,
        grid=(grid,),
        in_specs=[
            pl.BlockSpec((N_CORES, BLK, 16), lambda i: (0, i, 0)),
            pl.BlockSpec((BLK, m), lambda i: (i, 0)),
        ],
        out_specs=[
            pl.BlockSpec((BLK, 1), lambda i: (i, 0)),
            pl.BlockSpec((2, BLK, mh), lambda i: (0, i, 0)),
        ],
        out_shape=[
            jax.ShapeDtypeStruct((n, 1), jnp.float32),
            jax.ShapeDtypeStruct((2, n, mh), jnp.float32),
        ],
    )(hist, y)


def _tc_mid(p1, z1s, dvec, b1, w2p):
    _, n, mh = z1s.shape
    m = 2 * mh
    mp = w2p.shape[1]
    grid = (n + BLK - 1) // BLK

    def body(p_ref, z_ref, d_ref, b_ref, w_ref, o_ref):
        dv = d_ref[...]
        acc = p_ref[...] + z_ref[...]
        full = jnp.concatenate([acc[0], acc[1]], axis=1)
        h = dv * full + b_ref[...]
        h = jnp.maximum(h, 0.0)
        z2 = dv * jnp.dot(h, w_ref[...], preferred_element_type=jnp.float32)
        o_ref[0, :, :] = z2
        o_ref[1, :, :] = z2

    return pl.pallas_call(
        body,
        grid=(grid,),
        in_specs=[
            pl.BlockSpec((2, BLK, mh), lambda i: (0, i, 0)),
            pl.BlockSpec((2, BLK, mh), lambda i: (0, i, 0)),
            pl.BlockSpec((BLK, 1), lambda i: (i, 0)),
            pl.BlockSpec((1, m), lambda i: (0, 0)),
            pl.BlockSpec((m, mp), lambda i: (0, 0)),
        ],
        out_specs=pl.BlockSpec((2, BLK, mp), lambda i: (0, i, 0)),
        out_shape=jax.ShapeDtypeStruct((2, n, mp), jnp.float32),
    )(p1, z1s, dvec, b1, w2p)


def _tc_out(p2, z2, dvec, b2p):
    _, n, mp = z2.shape
    grid = (n + BLK - 1) // BLK

    def body(p_ref, z_ref, d_ref, b_ref, o_ref):
        o_ref[...] = d_ref[...] * (p_ref[0] + p_ref[1] + z_ref[0]) + b_ref[...]

    return pl.pallas_call(
        body,
        grid=(grid,),
        in_specs=[
            pl.BlockSpec((N_CORES, BLK, mp), lambda i: (0, i, 0)),
            pl.BlockSpec((1, BLK, mp), lambda i: (0, i, 0)),
            pl.BlockSpec((BLK, 1), lambda i: (i, 0)),
            pl.BlockSpec((1, mp), lambda i: (0, 0)),
        ],
        out_specs=pl.BlockSpec((BLK, mp), lambda i: (i, 0)),
        out_shape=jax.ShapeDtypeStruct((n, mp), jnp.float32),
    )(p2, z2, dvec, b2p)


def kernel(x, edge_index, W1, b1, W2, b2):
    n, nfeat = x.shape
    e = edge_index.shape[1]
    nclass = W2.shape[1]

    e_pad = _pad_up(e, 2 * NW * CHUNK)
    cpt_e = e_pad // (NW * CHUNK)
    cpt_f = 2 * cpt_e
    n_pad = _pad_up(n + 1, N_SUBCORES * CHUNK)

    row = edge_index[0]
    col = edge_index[1]
    pad = e_pad - e
    dummy = n + jnp.arange(pad, dtype=jnp.int32) % (n_pad - n)
    rowf = jnp.concatenate([row, dummy])
    colf = jnp.concatenate([col, jnp.zeros((pad,), jnp.int32)])
    rowp = rowf.reshape(NW, cpt_e, CHUNK)
    colp = colf.reshape(NW, cpt_e, CHUNK)
    sc1_off = jnp.where(jnp.arange(NW, dtype=jnp.int32) >= N_SUBCORES, n, 0)
    colp2 = colp + sc1_off[:, None, None]
    row16 = rowf.reshape(N_SUBCORES, cpt_f, CHUNK)
    col16 = colf.reshape(N_SUBCORES, cpt_f, CHUNK)
    rowp_f = jnp.concatenate([row16, row16], axis=0)
    colp_f = jnp.concatenate([col16, col16 + n], axis=0)

    ncp = _pad_up(nclass, 16)
    w2p = jnp.pad(W2, ((0, 0), (0, ncp - nclass)))
    b2p = jnp.pad(b2, (0, ncp - nclass)).reshape(1, ncp)

    hist = _sc_hist(rowp, n_pad, cpt_e)
    y1 = _tc_matmul(x, W1)
    dvec, z1s = _tc_scale(hist, y1)
    z1flat = z1s.reshape(2 * n, nfeat // 2)
    p1 = _sc_scatter(colp_f, rowp_f, z1flat, n_pad, cpt_f, nfeat // 2)
    z2s = _tc_mid(p1, z1s, dvec, b1.reshape(1, -1), w2p)
    p2 = _sc_scatter(colp2, rowp, z2s.reshape(2 * n, ncp), n_pad, cpt_e, ncp)
    out = _tc_out(p2, z2s, dvec, b2p)
    return out[:, :nclass]


# speedup vs baseline: 2.0289x; 2.0289x over previous
import functools

import jax
import jax.numpy as jnp
from jax import lax
from jax.experimental import pallas as pl
from jax.experimental.pallas import tpu as pltpu
from jax.experimental.pallas import tpu_sc as plsc

N_CORES = 2
N_SUBCORES = 16
NW = N_CORES * N_SUBCORES
CHUNK = 128
BLK = 512


def _pad_up(x, m):
    return (x + m - 1) // m * m


def _sc_mesh():
    return plsc.VectorSubcoreMesh(core_axis_name="c", subcore_axis_name="s")


_SC_PARAMS = pltpu.CompilerParams(use_tc_tiling_on_sc=False)


def _sc_hist(row_idx, n_pad, cpt):
    rows_per_tile = n_pad // N_SUBCORES
    slabs = rows_per_tile // CHUNK

    @functools.partial(
        pl.kernel,
        out_type=jax.ShapeDtypeStruct((N_CORES, n_pad, 16), jnp.float32),
        mesh=_sc_mesh(),
        scratch_types=[
            pltpu.VMEM((cpt, CHUNK), jnp.int32),
            pltpu.VMEM((CHUNK, 16), jnp.float32),
            pltpu.VMEM((CHUNK, 16), jnp.float32),
            pltpu.VMEM_SHARED((n_pad, 16), jnp.float32),
            pltpu.SemaphoreType.DMA,
        ],
        compiler_params=_SC_PARAMS,
    )
    def hist_kernel(row_hbm, out_hbm, idx_v, ones_v, zeros_v, hist_s, sem):
        c = lax.axis_index("c")
        s = lax.axis_index("s")
        w = c * N_SUBCORES + s
        pltpu.async_copy(row_hbm.at[w], idx_v, sem).wait()

        @pl.loop(0, CHUNK)
        def _(i):
            ones_v[i, :] = jnp.full((16,), 1.0, jnp.float32)
            zeros_v[i, :] = jnp.zeros((16,), jnp.float32)

        @pl.loop(0, slabs)
        def _(k):
            pltpu.sync_copy(
                zeros_v, hist_s.at[pl.ds(s * rows_per_tile + k * CHUNK, CHUNK)]
            )

        plsc.subcore_barrier()

        @pl.loop(0, cpt)
        def _(j):
            pltpu.sync_copy(ones_v, hist_s.at[idx_v.at[j]], add=True)

        plsc.subcore_barrier()
        pltpu.sync_copy(
            hist_s.at[pl.ds(s * rows_per_tile, rows_per_tile)],
            out_hbm.at[c].at[pl.ds(s * rows_per_tile, rows_per_tile)],
        )

    return hist_kernel(row_idx)


def _sc_scatter(col_idx, row_idx, z, n_pad, cpt, d):
    rows_per_tile = n_pad // N_SUBCORES
    slabs = rows_per_tile // CHUNK

    @functools.partial(
        pl.kernel,
        out_type=jax.ShapeDtypeStruct((N_CORES, n_pad, d), jnp.float32),
        mesh=_sc_mesh(),
        scratch_types=[
            pltpu.VMEM((cpt, CHUNK), jnp.int32),
            pltpu.VMEM((cpt, CHUNK), jnp.int32),
            pltpu.VMEM((CHUNK, d), jnp.float32),
            pltpu.VMEM((CHUNK, d), jnp.float32),
            pltpu.VMEM((CHUNK, d), jnp.float32),
            pltpu.VMEM_SHARED((n_pad, d), jnp.float32),
            pltpu.SemaphoreType.DMA,
            pltpu.SemaphoreType.DMA,
            pltpu.SemaphoreType.DMA,
        ],
        compiler_params=_SC_PARAMS,
    )
    def scat_kernel(col_hbm, row_hbm, z_hbm, out_hbm, cidx_v, ridx_v, buf_a,
                    buf_b, zeros_v, acc_s, sem_a, sem_b, sem):
        c = lax.axis_index("c")
        s = lax.axis_index("s")
        w = c * N_SUBCORES + s
        pltpu.async_copy(col_hbm.at[w], cidx_v, sem).wait()
        pltpu.async_copy(row_hbm.at[w], ridx_v, sem).wait()

        @pl.loop(0, CHUNK)
        def _(i):
            @pl.loop(0, d // 16)
            def _(t):
                zeros_v[i, pl.ds(t * 16, 16)] = jnp.zeros((16,), jnp.float32)

        @pl.loop(0, slabs)
        def _(k):
            pltpu.sync_copy(
                zeros_v, acc_s.at[pl.ds(s * rows_per_tile + k * CHUNK, CHUNK)]
            )

        plsc.subcore_barrier()

        pltpu.async_copy(z_hbm.at[cidx_v.at[0]], buf_a, sem_a)

        @pl.loop(0, cpt, step=2)
        def _(j):
            pltpu.async_copy(z_hbm.at[cidx_v.at[j + 1]], buf_b, sem_b)
            pltpu.make_async_copy(z_hbm.at[cidx_v.at[j]], buf_a, sem_a).wait()
            pltpu.sync_copy(buf_a, acc_s.at[ridx_v.at[j]], add=True)

            @pl.when(j + 2 < cpt)
            def _():
                pltpu.async_copy(z_hbm.at[cidx_v.at[j + 2]], buf_a, sem_a)

            pltpu.make_async_copy(z_hbm.at[cidx_v.at[j + 1]], buf_b, sem_b).wait()
            pltpu.sync_copy(buf_b, acc_s.at[ridx_v.at[j + 1]], add=True)

        plsc.subcore_barrier()
        pltpu.sync_copy(
            acc_s.at[pl.ds(s * rows_per_tile, rows_per_tile)],
            out_hbm.at[c].at[pl.ds(s * rows_per_tile, rows_per_tile)],
        )

    return scat_kernel(col_idx, row_idx, z)


def _tc_matmul(x, w):
    n, k = x.shape
    m = w.shape[1]
    grid = (n + BLK - 1) // BLK

    def body(x_ref, w_ref, o_ref):
        o_ref[...] = jnp.dot(x_ref[...], w_ref[...],
                             preferred_element_type=jnp.float32)

    return pl.pallas_call(
        body,
        grid=(grid,),
        in_specs=[
            pl.BlockSpec((BLK, k), lambda i: (i, 0)),
            pl.BlockSpec((k, m), lambda i: (0, 0)),
        ],
        out_specs=pl.BlockSpec((BLK, m), lambda i: (i, 0)),
        out_shape=jax.ShapeDtypeStruct((n, m), jnp.float32),
    )(x, w)


def _tc_scale(hist, y):
    n, m = y.shape
    mh = m // 2
    grid = (n + BLK - 1) // BLK

    def body(h_ref, y_ref, d_ref, z_ref):
        deg = h_ref[0, :, 0:1] + h_ref[1, :, 0:1] + 1.0
        dv = lax.rsqrt(deg)
        d_ref[...] = dv
        z = y_ref[...] * dv
        z_ref[0, :, :] = z[:, :mh]
        z_ref[1, :, :] = z[:, mh:]

    return pl.pallas_call(
        body,
        grid=(grid,),
        in_specs=[
            pl.BlockSpec((N_CORES, BLK, 16), lambda i: (0, i, 0)),
            pl.BlockSpec((BLK, m), lambda i: (i, 0)),
        ],
        out_specs=[
            pl.BlockSpec((BLK, 1), lambda i: (i, 0)),
            pl.BlockSpec((2, BLK, mh), lambda i: (0, i, 0)),
        ],
        out_shape=[
            jax.ShapeDtypeStruct((n, 1), jnp.float32),
            jax.ShapeDtypeStruct((2, n, mh), jnp.float32),
        ],
    )(hist, y)


def _tc_mid(p1, z1s, dvec, b1, w2p):
    _, n, mh = z1s.shape
    m = 2 * mh
    mp = w2p.shape[1]
    grid = (n + BLK - 1) // BLK

    def body(p_ref, z_ref, d_ref, b_ref, w_ref, o_ref):
        dv = d_ref[...]
        acc = p_ref[...] + z_ref[...]
        full = jnp.concatenate([acc[0], acc[1]], axis=1)
        h = dv * full + b_ref[...]
        h = jnp.maximum(h, 0.0)
        o_ref[...] = dv * jnp.dot(h, w_ref[...],
                                  preferred_element_type=jnp.float32)

    return pl.pallas_call(
        body,
        grid=(grid,),
        in_specs=[
            pl.BlockSpec((2, BLK, mh), lambda i: (0, i, 0)),
            pl.BlockSpec((2, BLK, mh), lambda i: (0, i, 0)),
            pl.BlockSpec((BLK, 1), lambda i: (i, 0)),
            pl.BlockSpec((1, m), lambda i: (0, 0)),
            pl.BlockSpec((m, mp), lambda i: (0, 0)),
        ],
        out_specs=pl.BlockSpec((BLK, mp), lambda i: (i, 0)),
        out_shape=jax.ShapeDtypeStruct((n, mp), jnp.float32),
    )(p1, z1s, dvec, b1, w2p)


def _tc_out(p2, z2, dvec, b2p):
    n, mp = z2.shape
    grid = (n + BLK - 1) // BLK

    def body(p_ref, z_ref, d_ref, b_ref, o_ref):
        o_ref[...] = d_ref[...] * (p_ref[0] + p_ref[1] + z_ref[...]) + b_ref[...]

    return pl.pallas_call(
        body,
        grid=(grid,),
        in_specs=[
            pl.BlockSpec((N_CORES, BLK, mp), lambda i: (0, i, 0)),
            pl.BlockSpec((BLK, mp), lambda i: (i, 0)),
            pl.BlockSpec((BLK, 1), lambda i: (i, 0)),
            pl.BlockSpec((1, mp), lambda i: (0, 0)),
        ],
        out_specs=pl.BlockSpec((BLK, mp), lambda i: (i, 0)),
        out_shape=jax.ShapeDtypeStruct((n, mp), jnp.float32),
    )(p2, z2, dvec, b2p)


def kernel(x, edge_index, W1, b1, W2, b2):
    n, nfeat = x.shape
    e = edge_index.shape[1]
    nclass = W2.shape[1]

    e_pad = _pad_up(e, 2 * NW * CHUNK)
    cpt_e = e_pad // (NW * CHUNK)
    cpt_f = 2 * cpt_e
    n_pad = _pad_up(n + 1, N_SUBCORES * CHUNK)

    row = edge_index[0]
    col = edge_index[1]
    pad = e_pad - e
    dummy = n + jnp.arange(pad, dtype=jnp.int32) % (n_pad - n)
    rowf = jnp.concatenate([row, dummy])
    dummy_c = (jnp.arange(pad, dtype=jnp.int32) * 131) % n
    colf = jnp.concatenate([col, dummy_c])
    rowp = rowf.reshape(NW, cpt_e, CHUNK)
    colp = colf.reshape(NW, cpt_e, CHUNK)
    row16 = rowf.reshape(N_SUBCORES, cpt_f, CHUNK)
    col16 = colf.reshape(N_SUBCORES, cpt_f, CHUNK)
    rowp_f = jnp.concatenate([row16, row16], axis=0)
    colp_f = jnp.concatenate([col16, col16 + n], axis=0)

    ncp = _pad_up(nclass, 16)
    w2p = jnp.pad(W2, ((0, 0), (0, ncp - nclass)))
    b2p = jnp.pad(b2, (0, ncp - nclass)).reshape(1, ncp)

    hist = _sc_hist(rowp, n_pad, cpt_e)
    y1 = _tc_matmul(x, W1)
    dvec, z1s = _tc_scale(hist, y1)
    z1flat = z1s.reshape(2 * n, nfeat // 2)
    p1 = _sc_scatter(colp_f, rowp_f, z1flat, n_pad, cpt_f, nfeat // 2)
    z2 = _tc_mid(p1, z1s, dvec, b1.reshape(1, -1), w2p)
    p2 = _sc_scatter(colp, rowp, z2, n_pad, cpt_e, ncp)
    out = _tc_out(p2, z2, dvec, b2p)
    return out[:, :nclass]


# speedup vs baseline: 2.2724x; 1.1200x over previous
import functools

import jax
import jax.numpy as jnp
from jax import lax
from jax.experimental import pallas as pl
from jax.experimental.pallas import tpu as pltpu
from jax.experimental.pallas import tpu_sc as plsc

N_CORES = 2
N_SUBCORES = 16
NW = N_CORES * N_SUBCORES
BLK = 512


def _pad_up(x, m):
    return (x + m - 1) // m * m


def _sc_mesh():
    return plsc.VectorSubcoreMesh(core_axis_name="c", subcore_axis_name="s")


_SC_PARAMS = pltpu.CompilerParams(use_tc_tiling_on_sc=False)


def _sc_hist(row_idx, n_pad, cpt):
    chunk = row_idx.shape[2]
    rows_per_tile = n_pad // N_SUBCORES
    slabs = rows_per_tile // chunk

    @functools.partial(
        pl.kernel,
        out_type=jax.ShapeDtypeStruct((N_CORES, n_pad, 16), jnp.float32),
        mesh=_sc_mesh(),
        scratch_types=[
            pltpu.VMEM((cpt, chunk), jnp.int32),
            pltpu.VMEM((chunk, 16), jnp.float32),
            pltpu.VMEM((chunk, 16), jnp.float32),
            pltpu.VMEM_SHARED((n_pad, 16), jnp.float32),
            pltpu.SemaphoreType.DMA,
        ],
        compiler_params=_SC_PARAMS,
    )
    def hist_kernel(row_hbm, out_hbm, idx_v, ones_v, zeros_v, hist_s, sem):
        c = lax.axis_index("c")
        s = lax.axis_index("s")
        w = c * N_SUBCORES + s
        pltpu.async_copy(row_hbm.at[w], idx_v, sem).wait()

        @pl.loop(0, chunk)
        def _(i):
            ones_v[i, :] = jnp.full((16,), 1.0, jnp.float32)
            zeros_v[i, :] = jnp.zeros((16,), jnp.float32)

        @pl.loop(0, slabs)
        def _(k):
            pltpu.sync_copy(
                zeros_v, hist_s.at[pl.ds(s * rows_per_tile + k * chunk, chunk)]
            )

        plsc.subcore_barrier()

        @pl.loop(0, cpt)
        def _(j):
            pltpu.sync_copy(ones_v, hist_s.at[idx_v.at[j]], add=True)

        plsc.subcore_barrier()
        pltpu.sync_copy(
            hist_s.at[pl.ds(s * rows_per_tile, rows_per_tile)],
            out_hbm.at[c].at[pl.ds(s * rows_per_tile, rows_per_tile)],
        )

    return hist_kernel(row_idx)


def _sc_scatter(col_idx, row_idx, z, n_pad, nbuf):
    _, cpt, chunk = col_idx.shape
    d = z.shape[1]
    rows_per_tile = n_pad // N_SUBCORES
    slabs = rows_per_tile // chunk

    @functools.partial(
        pl.kernel,
        out_type=jax.ShapeDtypeStruct((N_CORES, n_pad, d), jnp.float32),
        mesh=_sc_mesh(),
        scratch_types=[
            pltpu.VMEM((cpt, chunk), jnp.int32),
            pltpu.VMEM((cpt, chunk), jnp.int32),
            [pltpu.VMEM((chunk, d), jnp.float32) for _ in range(nbuf)],
            pltpu.VMEM((chunk, d), jnp.float32),
            pltpu.VMEM_SHARED((n_pad, d), jnp.float32),
            [pltpu.SemaphoreType.DMA for _ in range(nbuf)],
            pltpu.SemaphoreType.DMA,
        ],
        compiler_params=_SC_PARAMS,
    )
    def scat_kernel(col_hbm, row_hbm, z_hbm, out_hbm, cidx_v, ridx_v, bufs,
                    zeros_v, acc_s, sems, sem):
        c = lax.axis_index("c")
        s = lax.axis_index("s")
        w = c * N_SUBCORES + s
        pltpu.async_copy(col_hbm.at[w], cidx_v, sem).wait()
        pltpu.async_copy(row_hbm.at[w], ridx_v, sem).wait()

        @pl.loop(0, chunk)
        def _(i):
            @pl.loop(0, d // 16)
            def _(t):
                zeros_v[i, pl.ds(t * 16, 16)] = jnp.zeros((16,), jnp.float32)

        @pl.loop(0, slabs)
        def _(k):
            pltpu.sync_copy(
                zeros_v,
                acc_s.at[pl.ds(s * rows_per_tile + k * chunk, chunk)],
            )

        plsc.subcore_barrier()

        for b in range(nbuf):
            pltpu.async_copy(z_hbm.at[cidx_v.at[b]], bufs[b], sems[b])

        @pl.loop(0, cpt, step=nbuf)
        def _(j):
            for b in range(nbuf):
                pltpu.make_async_copy(
                    z_hbm.at[cidx_v.at[j + b]], bufs[b], sems[b]).wait()
                pltpu.sync_copy(
                    bufs[b], acc_s.at[ridx_v.at[j + b]], add=True)

                @pl.when(j + b + nbuf < cpt)
                def _():
                    pltpu.async_copy(
                        z_hbm.at[cidx_v.at[j + b + nbuf]], bufs[b], sems[b])

        plsc.subcore_barrier()
        pltpu.sync_copy(
            acc_s.at[pl.ds(s * rows_per_tile, rows_per_tile)],
            out_hbm.at[c].at[pl.ds(s * rows_per_tile, rows_per_tile)],
        )

    return scat_kernel(col_idx, row_idx, z)


def _tc_first(x, w1, hist):
    n, k = x.shape
    m = w1.shape[1]
    grid = (n + BLK - 1) // BLK

    def body(x_ref, w_ref, h_ref, d_ref, z_ref):
        y = jnp.dot(x_ref[...], w_ref[...], preferred_element_type=jnp.float32)
        deg = h_ref[0, :, 0:1] + h_ref[1, :, 0:1] + 1.0
        dv = lax.rsqrt(deg)
        d_ref[...] = dv
        z_ref[...] = y * dv

    return pl.pallas_call(
        body,
        grid=(grid,),
        in_specs=[
            pl.BlockSpec((BLK, k), lambda i: (i, 0)),
            pl.BlockSpec((k, m), lambda i: (0, 0)),
            pl.BlockSpec((N_CORES, BLK, 16), lambda i: (0, i, 0)),
        ],
        out_specs=[
            pl.BlockSpec((BLK, 1), lambda i: (i, 0)),
            pl.BlockSpec((BLK, m), lambda i: (i, 0)),
        ],
        out_shape=[
            jax.ShapeDtypeStruct((n, 1), jnp.float32),
            jax.ShapeDtypeStruct((n, m), jnp.float32),
        ],
    )(x, w1, hist)


def _tc_mid(p1, z1, dvec, b1, w2p):
    n, m = z1.shape
    mp = w2p.shape[1]
    grid = (n + BLK - 1) // BLK

    def body(p_ref, z_ref, d_ref, b_ref, w_ref, o_ref):
        dv = d_ref[...]
        h = dv * (p_ref[0] + p_ref[1] + z_ref[...]) + b_ref[...]
        h = jnp.maximum(h, 0.0)
        o_ref[...] = dv * jnp.dot(h, w_ref[...],
                                  preferred_element_type=jnp.float32)

    return pl.pallas_call(
        body,
        grid=(grid,),
        in_specs=[
            pl.BlockSpec((N_CORES, BLK, m), lambda i: (0, i, 0)),
            pl.BlockSpec((BLK, m), lambda i: (i, 0)),
            pl.BlockSpec((BLK, 1), lambda i: (i, 0)),
            pl.BlockSpec((1, m), lambda i: (0, 0)),
            pl.BlockSpec((m, mp), lambda i: (0, 0)),
        ],
        out_specs=pl.BlockSpec((BLK, mp), lambda i: (i, 0)),
        out_shape=jax.ShapeDtypeStruct((n, mp), jnp.float32),
    )(p1, z1, dvec, b1, w2p)


def _tc_out(p2, z2, dvec, b2, nclass):
    n, mp = z2.shape
    grid = (n + BLK - 1) // BLK

    def body(p_ref, z_ref, d_ref, b_ref, o_ref):
        full = d_ref[...] * (p_ref[0] + p_ref[1] + z_ref[...])
        o_ref[...] = full[:, :nclass] + b_ref[...]

    return pl.pallas_call(
        body,
        grid=(grid,),
        in_specs=[
            pl.BlockSpec((N_CORES, BLK, mp), lambda i: (0, i, 0)),
            pl.BlockSpec((BLK, mp), lambda i: (i, 0)),
            pl.BlockSpec((BLK, 1), lambda i: (i, 0)),
            pl.BlockSpec((1, nclass), lambda i: (0, 0)),
        ],
        out_specs=pl.BlockSpec((BLK, nclass), lambda i: (i, 0)),
        out_shape=jax.ShapeDtypeStruct((n, nclass), jnp.float32),
    )(p2, z2, dvec, b2)


def kernel(x, edge_index, W1, b1, W2, b2):
    n, nfeat = x.shape
    e = edge_index.shape[1]
    nclass = W2.shape[1]

    ch1 = 64
    ch2 = 128
    e_pad = _pad_up(e, NW * 2 * max(ch1 * 2, ch2 * 4))
    n_pad = _pad_up(n + 1, N_SUBCORES * 128)

    row = edge_index[0]
    col = edge_index[1]
    pad = e_pad - e

    dummy_r = n + jnp.arange(pad, dtype=jnp.int32) % (n_pad - n)
    dummy_c = (jnp.arange(pad, dtype=jnp.int32) * 131) % n
    rowf = jnp.concatenate([row, dummy_r])
    colf = jnp.concatenate([col, dummy_c])
    rowp1 = rowf.reshape(NW, e_pad // (NW * ch1), ch1)
    colp1 = colf.reshape(NW, e_pad // (NW * ch1), ch1)
    rowp2 = rowf.reshape(NW, e_pad // (NW * ch2), ch2)
    colp2 = colf.reshape(NW, e_pad // (NW * ch2), ch2)

    ncp = _pad_up(nclass, 16)
    w2p = jnp.pad(W2, ((0, 0), (0, ncp - nclass)))

    hist = _sc_hist(rowp2, n_pad, e_pad // (NW * ch2))
    dvec, z1 = _tc_first(x, W1, hist)
    p1 = _sc_scatter(colp1, rowp1, z1, n_pad, nbuf=2)
    z2 = _tc_mid(p1, z1, dvec, b1.reshape(1, -1), w2p)
    p2 = _sc_scatter(colp2, rowp2, z2, n_pad, nbuf=4)
    out = _tc_out(p2, z2, dvec, b2.reshape(1, -1), nclass)
    return out


# speedup vs baseline: 2.3649x; 1.0407x over previous
import functools

import jax
import jax.numpy as jnp
from jax import lax
from jax.experimental import pallas as pl
from jax.experimental.pallas import tpu as pltpu
from jax.experimental.pallas import tpu_sc as plsc

N_CORES = 2
N_SUBCORES = 16
NW = N_CORES * N_SUBCORES
BLK = 512


def _pad_up(x, m):
    return (x + m - 1) // m * m


def _sc_mesh():
    return plsc.VectorSubcoreMesh(core_axis_name="c", subcore_axis_name="s")


_SC_PARAMS = pltpu.CompilerParams(use_tc_tiling_on_sc=False)


def _sc_hist(row_idx, n_pad, cpt):
    chunk = row_idx.shape[2]
    rows_per_tile = n_pad // N_SUBCORES
    slabs = rows_per_tile // chunk

    @functools.partial(
        pl.kernel,
        out_type=jax.ShapeDtypeStruct((N_CORES, n_pad, 16), jnp.float32),
        mesh=_sc_mesh(),
        scratch_types=[
            pltpu.VMEM((cpt, chunk), jnp.int32),
            pltpu.VMEM((chunk, 16), jnp.float32),
            pltpu.VMEM((chunk, 16), jnp.float32),
            pltpu.VMEM_SHARED((n_pad, 16), jnp.float32),
            pltpu.SemaphoreType.DMA,
        ],
        compiler_params=_SC_PARAMS,
    )
    def hist_kernel(row_hbm, out_hbm, idx_v, ones_v, zeros_v, hist_s, sem):
        c = lax.axis_index("c")
        s = lax.axis_index("s")
        w = c * N_SUBCORES + s
        pltpu.async_copy(row_hbm.at[w], idx_v, sem).wait()

        @pl.loop(0, chunk)
        def _(i):
            ones_v[i, :] = jnp.full((16,), 1.0, jnp.float32)
            zeros_v[i, :] = jnp.zeros((16,), jnp.float32)

        @pl.loop(0, slabs)
        def _(k):
            pltpu.sync_copy(
                zeros_v, hist_s.at[pl.ds(s * rows_per_tile + k * chunk, chunk)]
            )

        plsc.subcore_barrier()

        @pl.loop(0, cpt)
        def _(j):
            pltpu.sync_copy(ones_v, hist_s.at[idx_v.at[j]], add=True)

        plsc.subcore_barrier()
        pltpu.sync_copy(
            hist_s.at[pl.ds(s * rows_per_tile, rows_per_tile)],
            out_hbm.at[c].at[pl.ds(s * rows_per_tile, rows_per_tile)],
        )

    return hist_kernel(row_idx)


def _sc_scatter(col_idx, row_idx, z, n_pad):
    _, cpt, chunk = col_idx.shape
    d = z.shape[1]
    rows_per_tile = n_pad // N_SUBCORES
    slabs = rows_per_tile // chunk

    @functools.partial(
        pl.kernel,
        out_type=jax.ShapeDtypeStruct((N_CORES, n_pad, d), jnp.float32),
        mesh=_sc_mesh(),
        scratch_types=[
            pltpu.VMEM((cpt, chunk), jnp.int32),
            pltpu.VMEM((cpt, chunk), jnp.int32),
            [pltpu.VMEM((chunk, d), jnp.float32) for _ in range(3)],
            pltpu.VMEM_SHARED((n_pad, d), jnp.float32),
            [pltpu.SemaphoreType.DMA for _ in range(3)],
            [pltpu.SemaphoreType.DMA for _ in range(3)],
            pltpu.SemaphoreType.DMA,
        ],
        compiler_params=_SC_PARAMS,
    )
    def scat_kernel(col_hbm, row_hbm, z_hbm, out_hbm, cidx_v, ridx_v, bufs,
                    acc_s, gsems, ssems, sem):
        c = lax.axis_index("c")
        s = lax.axis_index("s")
        w = c * N_SUBCORES + s
        pltpu.async_copy(col_hbm.at[w], cidx_v, sem).wait()
        pltpu.async_copy(row_hbm.at[w], ridx_v, sem).wait()

        zeros_v = bufs[2]

        @pl.loop(0, chunk)
        def _(i):
            @pl.loop(0, d // 16)
            def _(t):
                zeros_v[i, pl.ds(t * 16, 16)] = jnp.zeros((16,), jnp.float32)

        @pl.loop(0, slabs)
        def _(k):
            pltpu.sync_copy(
                zeros_v,
                acc_s.at[pl.ds(s * rows_per_tile + k * chunk, chunk)],
            )

        plsc.subcore_barrier()

        pltpu.async_copy(z_hbm.at[cidx_v.at[0]], bufs[0], gsems[0])
        pltpu.async_copy(z_hbm.at[cidx_v.at[1]], bufs[1], gsems[1])

        @pl.loop(0, cpt, step=3)
        def _(j):
            for t in range(3):
                k = j + t
                o = (t + 2) % 3
                pltpu.make_async_copy(
                    z_hbm.at[cidx_v.at[k]], bufs[t], gsems[t]).wait()
                pltpu.async_copy(
                    bufs[t], acc_s.at[ridx_v.at[k]], ssems[t], add=True)
                if t == 0:
                    @pl.when(k >= 1)
                    def _():
                        pltpu.make_async_copy(
                            bufs[o], acc_s.at[ridx_v.at[k - 1]],
                            ssems[o]).wait()
                    pltpu.async_copy(
                        z_hbm.at[cidx_v.at[k + 2]], bufs[o], gsems[o])
                else:
                    pltpu.make_async_copy(
                        bufs[o], acc_s.at[ridx_v.at[k - 1]], ssems[o]).wait()

                    @pl.when(k + 2 < cpt)
                    def _():
                        pltpu.async_copy(
                            z_hbm.at[cidx_v.at[k + 2]], bufs[o], gsems[o])

        pltpu.make_async_copy(
            bufs[(cpt - 1) % 3], acc_s.at[ridx_v.at[cpt - 1]],
            ssems[(cpt - 1) % 3]).wait()
        plsc.subcore_barrier()
        pltpu.sync_copy(
            acc_s.at[pl.ds(s * rows_per_tile, rows_per_tile)],
            out_hbm.at[c].at[pl.ds(s * rows_per_tile, rows_per_tile)],
        )

    return scat_kernel(col_idx, row_idx, z)


def _tc_first(x, w1, hist):
    n, k = x.shape
    m = w1.shape[1]
    grid = (n + BLK - 1) // BLK

    def body(x_ref, w_ref, h_ref, d_ref, z_ref):
        y = jnp.dot(x_ref[...], w_ref[...], preferred_element_type=jnp.float32)
        deg = h_ref[0, :, 0:1] + h_ref[1, :, 0:1] + 1.0
        dv = lax.rsqrt(deg)
        d_ref[...] = dv
        z_ref[...] = y * dv

    return pl.pallas_call(
        body,
        grid=(grid,),
        in_specs=[
            pl.BlockSpec((BLK, k), lambda i: (i, 0)),
            pl.BlockSpec((k, m), lambda i: (0, 0)),
            pl.BlockSpec((N_CORES, BLK, 16), lambda i: (0, i, 0)),
        ],
        out_specs=[
            pl.BlockSpec((BLK, 1), lambda i: (i, 0)),
            pl.BlockSpec((BLK, m), lambda i: (i, 0)),
        ],
        out_shape=[
            jax.ShapeDtypeStruct((n, 1), jnp.float32),
            jax.ShapeDtypeStruct((n, m), jnp.float32),
        ],
    )(x, w1, hist)


def _tc_mid(p1, z1, dvec, b1, w2p):
    n, m = z1.shape
    mp = w2p.shape[1]
    grid = (n + BLK - 1) // BLK

    def body(p_ref, z_ref, d_ref, b_ref, w_ref, o_ref):
        dv = d_ref[...]
        h = dv * (p_ref[0] + p_ref[1] + z_ref[...]) + b_ref[...]
        h = jnp.maximum(h, 0.0)
        o_ref[...] = dv * jnp.dot(h, w_ref[...],
                                  preferred_element_type=jnp.float32)

    return pl.pallas_call(
        body,
        grid=(grid,),
        in_specs=[
            pl.BlockSpec((N_CORES, BLK, m), lambda i: (0, i, 0)),
            pl.BlockSpec((BLK, m), lambda i: (i, 0)),
            pl.BlockSpec((BLK, 1), lambda i: (i, 0)),
            pl.BlockSpec((1, m), lambda i: (0, 0)),
            pl.BlockSpec((m, mp), lambda i: (0, 0)),
        ],
        out_specs=pl.BlockSpec((BLK, mp), lambda i: (i, 0)),
        out_shape=jax.ShapeDtypeStruct((n, mp), jnp.float32),
    )(p1, z1, dvec, b1, w2p)


def _tc_out(p2, z2, dvec, b2, nclass):
    n, mp = z2.shape
    grid = (n + BLK - 1) // BLK

    def body(p_ref, z_ref, d_ref, b_ref, o_ref):
        full = d_ref[...] * (p_ref[0] + p_ref[1] + z_ref[...])
        o_ref[...] = full[:, :nclass] + b_ref[...]

    return pl.pallas_call(
        body,
        grid=(grid,),
        in_specs=[
            pl.BlockSpec((N_CORES, BLK, mp), lambda i: (0, i, 0)),
            pl.BlockSpec((BLK, mp), lambda i: (i, 0)),
            pl.BlockSpec((BLK, 1), lambda i: (i, 0)),
            pl.BlockSpec((1, nclass), lambda i: (0, 0)),
        ],
        out_specs=pl.BlockSpec((BLK, nclass), lambda i: (i, 0)),
        out_shape=jax.ShapeDtypeStruct((n, nclass), jnp.float32),
    )(p2, z2, dvec, b2)


def kernel(x, edge_index, W1, b1, W2, b2):
    n, nfeat = x.shape
    e = edge_index.shape[1]
    nclass = W2.shape[1]

    ch1 = 64
    ch2 = 128
    e_pad = _pad_up(e, NW * 3 * max(ch1, ch2))
    n_pad = _pad_up(n + 1, N_SUBCORES * 128)

    row = edge_index[0]
    col = edge_index[1]
    pad = e_pad - e

    dummy_r = n + jnp.arange(pad, dtype=jnp.int32) % (n_pad - n)
    dummy_c = (jnp.arange(pad, dtype=jnp.int32) * 131) % n
    rowf = jnp.concatenate([row, dummy_r])
    colf = jnp.concatenate([col, dummy_c])
    rowp1 = rowf.reshape(NW, e_pad // (NW * ch1), ch1)
    colp1 = colf.reshape(NW, e_pad // (NW * ch1), ch1)
    rowp2 = rowf.reshape(NW, e_pad // (NW * ch2), ch2)
    colp2 = colf.reshape(NW, e_pad // (NW * ch2), ch2)

    ncp = _pad_up(nclass, 16)
    w2p = jnp.pad(W2, ((0, 0), (0, ncp - nclass)))

    hist = _sc_hist(rowp2, n_pad, e_pad // (NW * ch2))
    dvec, z1 = _tc_first(x, W1, hist)
    p1 = _sc_scatter(colp1, rowp1, z1, n_pad)
    z2 = _tc_mid(p1, z1, dvec, b1.reshape(1, -1), w2p)
    p2 = _sc_scatter(colp2, rowp2, z2, n_pad)
    out = _tc_out(p2, z2, dvec, b2.reshape(1, -1), nclass)
    return out


# speedup vs baseline: 2.5080x; 1.0605x over previous
import functools

import jax
import jax.numpy as jnp
from jax import lax
from jax.experimental import pallas as pl
from jax.experimental.pallas import tpu as pltpu
from jax.experimental.pallas import tpu_sc as plsc

N_CORES = 2
N_SUBCORES = 16
NW = N_CORES * N_SUBCORES
BLK = 1024


def _pad_up(x, m):
    return (x + m - 1) // m * m


def _sc_mesh():
    return plsc.VectorSubcoreMesh(core_axis_name="c", subcore_axis_name="s")


_SC_PARAMS = pltpu.CompilerParams(use_tc_tiling_on_sc=False)


def _sc_hist(row_idx, n_pad, cpt):
    chunk = row_idx.shape[2]
    rows_per_tile = n_pad // N_SUBCORES
    slabs = rows_per_tile // chunk

    @functools.partial(
        pl.kernel,
        out_type=jax.ShapeDtypeStruct((N_CORES, n_pad, 16), jnp.float32),
        mesh=_sc_mesh(),
        scratch_types=[
            pltpu.VMEM((cpt, chunk), jnp.int32),
            pltpu.VMEM((chunk, 16), jnp.float32),
            pltpu.VMEM((chunk, 16), jnp.float32),
            pltpu.VMEM_SHARED((n_pad, 16), jnp.float32),
            pltpu.SemaphoreType.DMA,
        ],
        compiler_params=_SC_PARAMS,
    )
    def hist_kernel(row_hbm, out_hbm, idx_v, ones_v, zeros_v, hist_s, sem):
        c = lax.axis_index("c")
        s = lax.axis_index("s")
        w = c * N_SUBCORES + s
        pltpu.async_copy(row_hbm.at[w], idx_v, sem).wait()

        @pl.loop(0, chunk)
        def _(i):
            ones_v[i, :] = jnp.full((16,), 1.0, jnp.float32)
            zeros_v[i, :] = jnp.zeros((16,), jnp.float32)

        @pl.loop(0, slabs)
        def _(k):
            pltpu.sync_copy(
                zeros_v, hist_s.at[pl.ds(s * rows_per_tile + k * chunk, chunk)]
            )

        plsc.subcore_barrier()

        @pl.loop(0, cpt)
        def _(j):
            pltpu.sync_copy(ones_v, hist_s.at[idx_v.at[j]], add=True)

        plsc.subcore_barrier()
        pltpu.sync_copy(
            hist_s.at[pl.ds(s * rows_per_tile, rows_per_tile)],
            out_hbm.at[c].at[pl.ds(s * rows_per_tile, rows_per_tile)],
        )

    return hist_kernel(row_idx)


def _sc_scatter(col_idx, row_idx, z, n_pad):
    _, cpt, chunk = col_idx.shape
    d = z.shape[1]
    rows_per_tile = n_pad // N_SUBCORES
    slabs = rows_per_tile // chunk

    @functools.partial(
        pl.kernel,
        out_type=jax.ShapeDtypeStruct((N_CORES, n_pad, d), jnp.float32),
        mesh=_sc_mesh(),
        scratch_types=[
            pltpu.VMEM((cpt, chunk), jnp.int32),
            pltpu.VMEM((cpt, chunk), jnp.int32),
            [pltpu.VMEM((chunk, d), jnp.float32) for _ in range(3)],
            pltpu.VMEM_SHARED((n_pad, d), jnp.float32),
            [pltpu.SemaphoreType.DMA for _ in range(3)],
            [pltpu.SemaphoreType.DMA for _ in range(3)],
            pltpu.SemaphoreType.DMA,
        ],
        compiler_params=_SC_PARAMS,
    )
    def scat_kernel(col_hbm, row_hbm, z_hbm, out_hbm, cidx_v, ridx_v, bufs,
                    acc_s, gsems, ssems, sem):
        c = lax.axis_index("c")
        s = lax.axis_index("s")
        w = c * N_SUBCORES + s
        pltpu.async_copy(col_hbm.at[w], cidx_v, sem).wait()
        pltpu.async_copy(row_hbm.at[w], ridx_v, sem).wait()

        zeros_v = bufs[2]

        @pl.loop(0, chunk)
        def _(i):
            @pl.loop(0, d // 16)
            def _(t):
                zeros_v[i, pl.ds(t * 16, 16)] = jnp.zeros((16,), jnp.float32)

        @pl.loop(0, slabs)
        def _(k):
            pltpu.sync_copy(
                zeros_v,
                acc_s.at[pl.ds(s * rows_per_tile + k * chunk, chunk)],
            )

        plsc.subcore_barrier()

        pltpu.async_copy(z_hbm.at[cidx_v.at[0]], bufs[0], gsems[0])
        pltpu.async_copy(z_hbm.at[cidx_v.at[1]], bufs[1], gsems[1])

        @pl.loop(0, cpt, step=3)
        def _(j):
            for t in range(3):
                k = j + t
                o = (t + 2) % 3
                pltpu.make_async_copy(
                    z_hbm.at[cidx_v.at[k]], bufs[t], gsems[t]).wait()
                pltpu.async_copy(
                    bufs[t], acc_s.at[ridx_v.at[k]], ssems[t], add=True)
                if t == 0:
                    @pl.when(k >= 1)
                    def _():
                        pltpu.make_async_copy(
                            bufs[o], acc_s.at[ridx_v.at[k - 1]],
                            ssems[o]).wait()
                    pltpu.async_copy(
                        z_hbm.at[cidx_v.at[k + 2]], bufs[o], gsems[o])
                else:
                    pltpu.make_async_copy(
                        bufs[o], acc_s.at[ridx_v.at[k - 1]], ssems[o]).wait()

                    @pl.when(k + 2 < cpt)
                    def _():
                        pltpu.async_copy(
                            z_hbm.at[cidx_v.at[k + 2]], bufs[o], gsems[o])

        pltpu.make_async_copy(
            bufs[(cpt - 1) % 3], acc_s.at[ridx_v.at[cpt - 1]],
            ssems[(cpt - 1) % 3]).wait()
        plsc.subcore_barrier()
        pltpu.sync_copy(
            acc_s.at[pl.ds(s * rows_per_tile, rows_per_tile)],
            out_hbm.at[c].at[pl.ds(s * rows_per_tile, rows_per_tile)],
        )

    return scat_kernel(col_idx, row_idx, z)


def _tc_first(x, w1, hist):
    n, k = x.shape
    m = w1.shape[1]
    grid = (n + BLK - 1) // BLK

    def body(x_ref, w_ref, h_ref, d_ref, z_ref):
        y = jnp.dot(x_ref[...], w_ref[...], preferred_element_type=jnp.float32)
        deg = h_ref[0, :, 0:1] + h_ref[1, :, 0:1] + 1.0
        dv = lax.rsqrt(deg)
        d_ref[...] = dv
        z_ref[...] = y * dv

    return pl.pallas_call(
        body,
        grid=(grid,),
        in_specs=[
            pl.BlockSpec((BLK, k), lambda i: (i, 0)),
            pl.BlockSpec((k, m), lambda i: (0, 0)),
            pl.BlockSpec((N_CORES, BLK, 16), lambda i: (0, i, 0)),
        ],
        out_specs=[
            pl.BlockSpec((BLK, 1), lambda i: (i, 0)),
            pl.BlockSpec((BLK, m), lambda i: (i, 0)),
        ],
        out_shape=[
            jax.ShapeDtypeStruct((n, 1), jnp.float32),
            jax.ShapeDtypeStruct((n, m), jnp.float32),
        ],
    )(x, w1, hist)


def _tc_mid(p1, z1, dvec, b1, w2p):
    n, m = z1.shape
    mp = w2p.shape[1]
    grid = (n + BLK - 1) // BLK

    def body(p_ref, z_ref, d_ref, b_ref, w_ref, o_ref):
        dv = d_ref[...]
        h = dv * (p_ref[0] + p_ref[1] + z_ref[...]) + b_ref[...]
        h = jnp.maximum(h, 0.0)
        o_ref[...] = dv * jnp.dot(h, w_ref[...],
                                  preferred_element_type=jnp.float32)

    return pl.pallas_call(
        body,
        grid=(grid,),
        in_specs=[
            pl.BlockSpec((N_CORES, BLK, m), lambda i: (0, i, 0)),
            pl.BlockSpec((BLK, m), lambda i: (i, 0)),
            pl.BlockSpec((BLK, 1), lambda i: (i, 0)),
            pl.BlockSpec((1, m), lambda i: (0, 0)),
            pl.BlockSpec((m, mp), lambda i: (0, 0)),
        ],
        out_specs=pl.BlockSpec((BLK, mp), lambda i: (i, 0)),
        out_shape=jax.ShapeDtypeStruct((n, mp), jnp.float32),
    )(p1, z1, dvec, b1, w2p)


def _tc_out(p2, z2, dvec, b2, nclass):
    n, mp = z2.shape
    grid = (n + BLK - 1) // BLK

    def body(p_ref, z_ref, d_ref, b_ref, o_ref):
        full = d_ref[...] * (p_ref[0] + p_ref[1] + z_ref[...])
        o_ref[...] = full[:, :nclass] + b_ref[...]

    return pl.pallas_call(
        body,
        grid=(grid,),
        in_specs=[
            pl.BlockSpec((N_CORES, BLK, mp), lambda i: (0, i, 0)),
            pl.BlockSpec((BLK, mp), lambda i: (i, 0)),
            pl.BlockSpec((BLK, 1), lambda i: (i, 0)),
            pl.BlockSpec((1, nclass), lambda i: (0, 0)),
        ],
        out_specs=pl.BlockSpec((BLK, nclass), lambda i: (i, 0)),
        out_shape=jax.ShapeDtypeStruct((n, nclass), jnp.float32),
    )(p2, z2, dvec, b2)


def kernel(x, edge_index, W1, b1, W2, b2):
    n, nfeat = x.shape
    e = edge_index.shape[1]
    nclass = W2.shape[1]

    ch1 = 64
    ch2 = 128
    e_pad = _pad_up(e, NW * 3 * max(ch1, ch2))
    n_pad = _pad_up(n + 1, N_SUBCORES * 128)

    row = edge_index[0]
    col = edge_index[1]
    pad = e_pad - e

    dummy_r = n + jnp.arange(pad, dtype=jnp.int32) % (n_pad - n)
    dummy_c = (jnp.arange(pad, dtype=jnp.int32) * 131) % n
    rowf = jnp.concatenate([row, dummy_r])
    colf = jnp.concatenate([col, dummy_c])
    rowp1 = rowf.reshape(NW, e_pad // (NW * ch1), ch1)
    colp1 = colf.reshape(NW, e_pad // (NW * ch1), ch1)
    rowp2 = rowf.reshape(NW, e_pad // (NW * ch2), ch2)
    colp2 = colf.reshape(NW, e_pad // (NW * ch2), ch2)

    ncp = _pad_up(nclass, 16)
    w2p = jnp.pad(W2, ((0, 0), (0, ncp - nclass)))

    hist = _sc_hist(rowp2, n_pad, e_pad // (NW * ch2))
    dvec, z1 = _tc_first(x, W1, hist)
    p1 = _sc_scatter(colp1, rowp1, z1, n_pad)
    z2 = _tc_mid(p1, z1, dvec, b1.reshape(1, -1), w2p)
    p2 = _sc_scatter(colp2, rowp2, z2, n_pad)
    out = _tc_out(p2, z2, dvec, b2.reshape(1, -1), nclass)
    return out


# speedup vs baseline: 2.5632x; 1.0220x over previous
import functools

import jax
import jax.numpy as jnp
from jax import lax
from jax.experimental import pallas as pl
from jax.experimental.pallas import tpu as pltpu
from jax.experimental.pallas import tpu_sc as plsc

N_CORES = 2
N_SUBCORES = 16
NW = N_CORES * N_SUBCORES
BLK = 2048


def _pad_up(x, m):
    return (x + m - 1) // m * m


def _sc_mesh():
    return plsc.VectorSubcoreMesh(core_axis_name="c", subcore_axis_name="s")


_SC_PARAMS = pltpu.CompilerParams(use_tc_tiling_on_sc=False)


def _sc_hist(row_idx, n_pad, cpt):
    chunk = row_idx.shape[2]
    rows_per_tile = n_pad // N_SUBCORES
    slabs = rows_per_tile // chunk

    @functools.partial(
        pl.kernel,
        out_type=jax.ShapeDtypeStruct((N_CORES, n_pad, 16), jnp.float32),
        mesh=_sc_mesh(),
        scratch_types=[
            pltpu.VMEM((cpt, chunk), jnp.int32),
            pltpu.VMEM((chunk, 16), jnp.float32),
            pltpu.VMEM((chunk, 16), jnp.float32),
            pltpu.VMEM_SHARED((n_pad, 16), jnp.float32),
            pltpu.SemaphoreType.DMA,
        ],
        compiler_params=_SC_PARAMS,
    )
    def hist_kernel(row_hbm, out_hbm, idx_v, ones_v, zeros_v, hist_s, sem):
        c = lax.axis_index("c")
        s = lax.axis_index("s")
        w = c * N_SUBCORES + s
        pltpu.async_copy(row_hbm.at[w], idx_v, sem).wait()

        @pl.loop(0, chunk)
        def _(i):
            ones_v[i, :] = jnp.full((16,), 1.0, jnp.float32)
            zeros_v[i, :] = jnp.zeros((16,), jnp.float32)

        @pl.loop(0, slabs)
        def _(k):
            pltpu.sync_copy(
                zeros_v, hist_s.at[pl.ds(s * rows_per_tile + k * chunk, chunk)]
            )

        plsc.subcore_barrier()

        @pl.loop(0, cpt)
        def _(j):
            pltpu.sync_copy(ones_v, hist_s.at[idx_v.at[j]], add=True)

        plsc.subcore_barrier()
        pltpu.sync_copy(
            hist_s.at[pl.ds(s * rows_per_tile, rows_per_tile)],
            out_hbm.at[c].at[pl.ds(s * rows_per_tile, rows_per_tile)],
        )

    return hist_kernel(row_idx)


def _sc_scatter(col_idx, row_idx, z, n_pad):
    _, cpt, chunk = col_idx.shape
    d = z.shape[1]
    rows_per_tile = n_pad // N_SUBCORES
    slabs = rows_per_tile // chunk

    @functools.partial(
        pl.kernel,
        out_type=jax.ShapeDtypeStruct((N_CORES, n_pad, d), jnp.float32),
        mesh=_sc_mesh(),
        scratch_types=[
            pltpu.VMEM((cpt, chunk), jnp.int32),
            pltpu.VMEM((cpt, chunk), jnp.int32),
            [pltpu.VMEM((chunk, d), jnp.float32) for _ in range(3)],
            pltpu.VMEM_SHARED((n_pad, d), jnp.float32),
            [pltpu.SemaphoreType.DMA for _ in range(3)],
            [pltpu.SemaphoreType.DMA for _ in range(3)],
            pltpu.SemaphoreType.DMA,
        ],
        compiler_params=_SC_PARAMS,
    )
    def scat_kernel(col_hbm, row_hbm, z_hbm, out_hbm, cidx_v, ridx_v, bufs,
                    acc_s, gsems, ssems, sem):
        c = lax.axis_index("c")
        s = lax.axis_index("s")
        w = c * N_SUBCORES + s
        pltpu.async_copy(col_hbm.at[w], cidx_v, sem).wait()
        pltpu.async_copy(row_hbm.at[w], ridx_v, sem).wait()

        zeros_v = bufs[2]

        @pl.loop(0, chunk)
        def _(i):
            @pl.loop(0, d // 16)
            def _(t):
                zeros_v[i, pl.ds(t * 16, 16)] = jnp.zeros((16,), jnp.float32)

        @pl.loop(0, slabs)
        def _(k):
            pltpu.sync_copy(
                zeros_v,
                acc_s.at[pl.ds(s * rows_per_tile + k * chunk, chunk)],
            )

        plsc.subcore_barrier()

        pltpu.async_copy(z_hbm.at[cidx_v.at[0]], bufs[0], gsems[0])
        pltpu.async_copy(z_hbm.at[cidx_v.at[1]], bufs[1], gsems[1])

        @pl.loop(0, cpt, step=3)
        def _(j):
            for t in range(3):
                k = j + t
                o = (t + 2) % 3
                pltpu.make_async_copy(
                    z_hbm.at[cidx_v.at[k]], bufs[t], gsems[t]).wait()
                pltpu.async_copy(
                    bufs[t], acc_s.at[ridx_v.at[k]], ssems[t], add=True)
                if t == 0:
                    @pl.when(k >= 1)
                    def _():
                        pltpu.make_async_copy(
                            bufs[o], acc_s.at[ridx_v.at[k - 1]],
                            ssems[o]).wait()
                    pltpu.async_copy(
                        z_hbm.at[cidx_v.at[k + 2]], bufs[o], gsems[o])
                else:
                    pltpu.make_async_copy(
                        bufs[o], acc_s.at[ridx_v.at[k - 1]], ssems[o]).wait()

                    @pl.when(k + 2 < cpt)
                    def _():
                        pltpu.async_copy(
                            z_hbm.at[cidx_v.at[k + 2]], bufs[o], gsems[o])

        pltpu.make_async_copy(
            bufs[(cpt - 1) % 3], acc_s.at[ridx_v.at[cpt - 1]],
            ssems[(cpt - 1) % 3]).wait()
        plsc.subcore_barrier()
        pltpu.sync_copy(
            acc_s.at[pl.ds(s * rows_per_tile, rows_per_tile)],
            out_hbm.at[c].at[pl.ds(s * rows_per_tile, rows_per_tile)],
        )

    return scat_kernel(col_idx, row_idx, z)


def _tc_first(x, w1, hist):
    n, k = x.shape
    m = w1.shape[1]
    grid = (n + BLK - 1) // BLK

    def body(x_ref, w_ref, h_ref, d_ref, z_ref):
        y = jnp.dot(x_ref[...], w_ref[...], preferred_element_type=jnp.float32)
        deg = h_ref[0, :, 0:1] + h_ref[1, :, 0:1] + 1.0
        dv = lax.rsqrt(deg)
        d_ref[...] = dv
        z_ref[...] = y * dv

    return pl.pallas_call(
        body,
        grid=(grid,),
        in_specs=[
            pl.BlockSpec((BLK, k), lambda i: (i, 0)),
            pl.BlockSpec((k, m), lambda i: (0, 0)),
            pl.BlockSpec((N_CORES, BLK, 16), lambda i: (0, i, 0)),
        ],
        out_specs=[
            pl.BlockSpec((BLK, 1), lambda i: (i, 0)),
            pl.BlockSpec((BLK, m), lambda i: (i, 0)),
        ],
        out_shape=[
            jax.ShapeDtypeStruct((n, 1), jnp.float32),
            jax.ShapeDtypeStruct((n, m), jnp.float32),
        ],
    )(x, w1, hist)


def _tc_mid(p1, z1, dvec, b1, w2p):
    n, m = z1.shape
    mp = w2p.shape[1]
    grid = (n + BLK - 1) // BLK

    def body(p_ref, z_ref, d_ref, b_ref, w_ref, o_ref):
        dv = d_ref[...]
        h = dv * (p_ref[0] + p_ref[1] + z_ref[...]) + b_ref[...]
        h = jnp.maximum(h, 0.0)
        o_ref[...] = dv * jnp.dot(h, w_ref[...],
                                  preferred_element_type=jnp.float32)

    return pl.pallas_call(
        body,
        grid=(grid,),
        in_specs=[
            pl.BlockSpec((N_CORES, BLK, m), lambda i: (0, i, 0)),
            pl.BlockSpec((BLK, m), lambda i: (i, 0)),
            pl.BlockSpec((BLK, 1), lambda i: (i, 0)),
            pl.BlockSpec((1, m), lambda i: (0, 0)),
            pl.BlockSpec((m, mp), lambda i: (0, 0)),
        ],
        out_specs=pl.BlockSpec((BLK, mp), lambda i: (i, 0)),
        out_shape=jax.ShapeDtypeStruct((n, mp), jnp.float32),
    )(p1, z1, dvec, b1, w2p)


def _tc_out(p2, z2, dvec, b2, nclass):
    n, mp = z2.shape
    grid = (n + BLK - 1) // BLK

    def body(p_ref, z_ref, d_ref, b_ref, o_ref):
        full = d_ref[...] * (p_ref[0] + p_ref[1] + z_ref[...])
        o_ref[...] = full[:, :nclass] + b_ref[...]

    return pl.pallas_call(
        body,
        grid=(grid,),
        in_specs=[
            pl.BlockSpec((N_CORES, BLK, mp), lambda i: (0, i, 0)),
            pl.BlockSpec((BLK, mp), lambda i: (i, 0)),
            pl.BlockSpec((BLK, 1), lambda i: (i, 0)),
            pl.BlockSpec((1, nclass), lambda i: (0, 0)),
        ],
        out_specs=pl.BlockSpec((BLK, nclass), lambda i: (i, 0)),
        out_shape=jax.ShapeDtypeStruct((n, nclass), jnp.float32),
    )(p2, z2, dvec, b2)


def kernel(x, edge_index, W1, b1, W2, b2):
    n, nfeat = x.shape
    e = edge_index.shape[1]
    nclass = W2.shape[1]

    ch1 = 64
    ch2 = 128
    e_pad = _pad_up(e, NW * 3 * max(ch1, ch2))
    n_pad = _pad_up(n + 1, N_SUBCORES * 128)

    row = edge_index[0]
    col = edge_index[1]
    pad = e_pad - e

    dummy_r = n + jnp.arange(pad, dtype=jnp.int32) % (n_pad - n)
    dummy_c = (jnp.arange(pad, dtype=jnp.int32) * 131) % n
    rowf = jnp.concatenate([row, dummy_r])
    colf = jnp.concatenate([col, dummy_c])
    rowp1 = rowf.reshape(NW, e_pad // (NW * ch1), ch1)
    colp1 = colf.reshape(NW, e_pad // (NW * ch1), ch1)
    rowp2 = rowf.reshape(NW, e_pad // (NW * ch2), ch2)
    colp2 = colf.reshape(NW, e_pad // (NW * ch2), ch2)

    ncp = _pad_up(nclass, 16)
    w2p = jnp.pad(W2, ((0, 0), (0, ncp - nclass)))

    hist = _sc_hist(rowp2, n_pad, e_pad // (NW * ch2))
    dvec, z1 = _tc_first(x, W1, hist)
    p1 = _sc_scatter(colp1, rowp1, z1, n_pad)
    z2 = _tc_mid(p1, z1, dvec, b1.reshape(1, -1), w2p)
    p2 = _sc_scatter(colp2, rowp2, z2, n_pad)
    out = _tc_out(p2, z2, dvec, b2.reshape(1, -1), nclass)
    return out


# speedup vs baseline: 2.5961x; 1.0128x over previous
import functools

import jax
import jax.numpy as jnp
from jax import lax
from jax.experimental import pallas as pl
from jax.experimental.pallas import tpu as pltpu
from jax.experimental.pallas import tpu_sc as plsc

N_CORES = 2
N_SUBCORES = 16
NW = N_CORES * N_SUBCORES
BLK = 2048


def _pad_up(x, m):
    return (x + m - 1) // m * m


def _sc_mesh():
    return plsc.VectorSubcoreMesh(core_axis_name="c", subcore_axis_name="s")


_SC_PARAMS = pltpu.CompilerParams(use_tc_tiling_on_sc=False)


def _sc_hist(row_idx, n_pad, cpt):
    chunk = row_idx.shape[2]
    rows_per_tile = n_pad // N_SUBCORES
    slabs = rows_per_tile // chunk

    @functools.partial(
        pl.kernel,
        out_type=jax.ShapeDtypeStruct((N_CORES, n_pad, 16), jnp.float32),
        mesh=_sc_mesh(),
        scratch_types=[
            pltpu.VMEM((cpt, chunk), jnp.int32),
            pltpu.VMEM((chunk, 16), jnp.float32),
            pltpu.VMEM((chunk, 16), jnp.float32),
            pltpu.VMEM_SHARED((n_pad, 16), jnp.float32),
            pltpu.SemaphoreType.DMA,
        ],
        compiler_params=_SC_PARAMS,
    )
    def hist_kernel(row_hbm, out_hbm, idx_v, ones_v, zeros_v, hist_s, sem):
        c = lax.axis_index("c")
        s = lax.axis_index("s")
        w = c * N_SUBCORES + s
        pltpu.async_copy(row_hbm.at[w], idx_v, sem).wait()

        @pl.loop(0, chunk)
        def _(i):
            ones_v[i, :] = jnp.full((16,), 1.0, jnp.float32)
            zeros_v[i, :] = jnp.zeros((16,), jnp.float32)

        @pl.loop(0, slabs)
        def _(k):
            pltpu.sync_copy(
                zeros_v, hist_s.at[pl.ds(s * rows_per_tile + k * chunk, chunk)]
            )

        plsc.subcore_barrier()

        @pl.loop(0, cpt)
        def _(j):
            pltpu.sync_copy(ones_v, hist_s.at[idx_v.at[j]], add=True)

        plsc.subcore_barrier()
        pltpu.sync_copy(
            hist_s.at[pl.ds(s * rows_per_tile, rows_per_tile)],
            out_hbm.at[c].at[pl.ds(s * rows_per_tile, rows_per_tile)],
        )

    return hist_kernel(row_idx)


def _sc_scatter(col_idx, row_idx, z, n_pad, sync_ring=False):
    _, cpt, chunk = col_idx.shape
    d = z.shape[1]
    rows_per_tile = n_pad // N_SUBCORES
    slabs = rows_per_tile // chunk

    @functools.partial(
        pl.kernel,
        out_type=jax.ShapeDtypeStruct((N_CORES, n_pad, d), jnp.float32),
        mesh=_sc_mesh(),
        scratch_types=[
            pltpu.VMEM((cpt, chunk), jnp.int32),
            pltpu.VMEM((cpt, chunk), jnp.int32),
            [pltpu.VMEM((chunk, d), jnp.float32) for _ in range(3)],
            pltpu.VMEM_SHARED((n_pad, d), jnp.float32),
            [pltpu.SemaphoreType.DMA for _ in range(3)],
            [pltpu.SemaphoreType.DMA for _ in range(3)],
            pltpu.SemaphoreType.DMA,
        ],
        compiler_params=_SC_PARAMS,
    )
    def scat_kernel(col_hbm, row_hbm, z_hbm, out_hbm, cidx_v, ridx_v, bufs,
                    acc_s, gsems, ssems, sem):
        c = lax.axis_index("c")
        s = lax.axis_index("s")
        w = c * N_SUBCORES + s
        pltpu.async_copy(col_hbm.at[w], cidx_v, sem).wait()
        pltpu.async_copy(row_hbm.at[w], ridx_v, sem).wait()

        zeros_v = bufs[2]

        @pl.loop(0, chunk)
        def _(i):
            @pl.loop(0, d // 16)
            def _(t):
                zeros_v[i, pl.ds(t * 16, 16)] = jnp.zeros((16,), jnp.float32)

        @pl.loop(0, slabs)
        def _(k):
            pltpu.sync_copy(
                zeros_v,
                acc_s.at[pl.ds(s * rows_per_tile + k * chunk, chunk)],
            )

        plsc.subcore_barrier()

        if sync_ring:
            for b in range(3):
                pltpu.async_copy(z_hbm.at[cidx_v.at[b]], bufs[b], gsems[b])

            @pl.loop(0, cpt, step=3)
            def _(j):
                for t in range(3):
                    pltpu.make_async_copy(
                        z_hbm.at[cidx_v.at[j + t]], bufs[t], gsems[t]).wait()
                    pltpu.sync_copy(
                        bufs[t], acc_s.at[ridx_v.at[j + t]], add=True)

                    @pl.when(j + t + 3 < cpt)
                    def _():
                        pltpu.async_copy(
                            z_hbm.at[cidx_v.at[j + t + 3]], bufs[t],
                            gsems[t])
        else:
            pltpu.async_copy(z_hbm.at[cidx_v.at[0]], bufs[0], gsems[0])
            pltpu.async_copy(z_hbm.at[cidx_v.at[1]], bufs[1], gsems[1])

            @pl.loop(0, cpt, step=3)
            def _(j):
                for t in range(3):
                    k = j + t
                    o = (t + 2) % 3
                    pltpu.make_async_copy(
                        z_hbm.at[cidx_v.at[k]], bufs[t], gsems[t]).wait()
                    pltpu.async_copy(
                        bufs[t], acc_s.at[ridx_v.at[k]], ssems[t], add=True)
                    if t == 0:
                        @pl.when(k >= 1)
                        def _():
                            pltpu.make_async_copy(
                                bufs[o], acc_s.at[ridx_v.at[k - 1]],
                                ssems[o]).wait()
                        pltpu.async_copy(
                            z_hbm.at[cidx_v.at[k + 2]], bufs[o], gsems[o])
                    else:
                        pltpu.make_async_copy(
                            bufs[o], acc_s.at[ridx_v.at[k - 1]],
                            ssems[o]).wait()

                        @pl.when(k + 2 < cpt)
                        def _():
                            pltpu.async_copy(
                                z_hbm.at[cidx_v.at[k + 2]], bufs[o],
                                gsems[o])

            pltpu.make_async_copy(
                bufs[(cpt - 1) % 3], acc_s.at[ridx_v.at[cpt - 1]],
                ssems[(cpt - 1) % 3]).wait()
        plsc.subcore_barrier()
        pltpu.sync_copy(
            acc_s.at[pl.ds(s * rows_per_tile, rows_per_tile)],
            out_hbm.at[c].at[pl.ds(s * rows_per_tile, rows_per_tile)],
        )

    return scat_kernel(col_idx, row_idx, z)


def _tc_matmul(x, w1):
    n, k = x.shape
    m = w1.shape[1]
    grid = (n + BLK - 1) // BLK

    def body(x_ref, w_ref, o_ref):
        o_ref[...] = jnp.dot(x_ref[...], w_ref[...],
                             preferred_element_type=jnp.float32)

    return pl.pallas_call(
        body,
        grid=(grid,),
        in_specs=[
            pl.BlockSpec((BLK, k), lambda i: (i, 0)),
            pl.BlockSpec((k, m), lambda i: (0, 0)),
        ],
        out_specs=pl.BlockSpec((BLK, m), lambda i: (i, 0)),
        out_shape=jax.ShapeDtypeStruct((n, m), jnp.float32),
    )(x, w1)


def _tc_scale(hist, y):
    n, m = y.shape
    grid = (n + BLK - 1) // BLK

    def body(h_ref, y_ref, d_ref, z_ref):
        deg = h_ref[0, :, 0:1] + h_ref[1, :, 0:1] + 1.0
        dv = lax.rsqrt(deg)
        d_ref[...] = dv
        z_ref[...] = y_ref[...] * dv

    return pl.pallas_call(
        body,
        grid=(grid,),
        in_specs=[
            pl.BlockSpec((N_CORES, BLK, 16), lambda i: (0, i, 0)),
            pl.BlockSpec((BLK, m), lambda i: (i, 0)),
        ],
        out_specs=[
            pl.BlockSpec((BLK, 1), lambda i: (i, 0)),
            pl.BlockSpec((BLK, m), lambda i: (i, 0)),
        ],
        out_shape=[
            jax.ShapeDtypeStruct((n, 1), jnp.float32),
            jax.ShapeDtypeStruct((n, m), jnp.float32),
        ],
    )(hist, y)


def _tc_mid(p1, z1, dvec, b1, w2p):
    n, m = z1.shape
    mp = w2p.shape[1]
    grid = (n + BLK - 1) // BLK

    def body(p_ref, z_ref, d_ref, b_ref, w_ref, o_ref):
        dv = d_ref[...]
        h = dv * (p_ref[0] + p_ref[1] + z_ref[...]) + b_ref[...]
        h = jnp.maximum(h, 0.0)
        o_ref[...] = dv * jnp.dot(h, w_ref[...],
                                  preferred_element_type=jnp.float32)

    return pl.pallas_call(
        body,
        grid=(grid,),
        in_specs=[
            pl.BlockSpec((N_CORES, BLK, m), lambda i: (0, i, 0)),
            pl.BlockSpec((BLK, m), lambda i: (i, 0)),
            pl.BlockSpec((BLK, 1), lambda i: (i, 0)),
            pl.BlockSpec((1, m), lambda i: (0, 0)),
            pl.BlockSpec((m, mp), lambda i: (0, 0)),
        ],
        out_specs=pl.BlockSpec((BLK, mp), lambda i: (i, 0)),
        out_shape=jax.ShapeDtypeStruct((n, mp), jnp.float32),
    )(p1, z1, dvec, b1, w2p)


def _tc_out(p2, z2, dvec, b2, nclass):
    n, mp = z2.shape
    grid = (n + BLK - 1) // BLK

    def body(p_ref, z_ref, d_ref, b_ref, o_ref):
        full = d_ref[...] * (p_ref[0] + p_ref[1] + z_ref[...])
        o_ref[...] = full[:, :nclass] + b_ref[...]

    return pl.pallas_call(
        body,
        grid=(grid,),
        in_specs=[
            pl.BlockSpec((N_CORES, BLK, mp), lambda i: (0, i, 0)),
            pl.BlockSpec((BLK, mp), lambda i: (i, 0)),
            pl.BlockSpec((BLK, 1), lambda i: (i, 0)),
            pl.BlockSpec((1, nclass), lambda i: (0, 0)),
        ],
        out_specs=pl.BlockSpec((BLK, nclass), lambda i: (i, 0)),
        out_shape=jax.ShapeDtypeStruct((n, nclass), jnp.float32),
    )(p2, z2, dvec, b2)


def kernel(x, edge_index, W1, b1, W2, b2):
    n, nfeat = x.shape
    e = edge_index.shape[1]
    nclass = W2.shape[1]

    ch1 = 64
    ch2 = 128
    e_pad = _pad_up(e, NW * 3 * max(ch1, ch2))
    n_pad = _pad_up(n + 1, N_SUBCORES * 128)

    row = edge_index[0]
    col = edge_index[1]
    pad = e_pad - e

    dummy_r = n + jnp.arange(pad, dtype=jnp.int32) % (n_pad - n)
    dummy_c = (jnp.arange(pad, dtype=jnp.int32) * 131) % n
    rowf = jnp.concatenate([row, dummy_r])
    colf = jnp.concatenate([col, dummy_c])
    rowp1 = rowf.reshape(NW, e_pad // (NW * ch1), ch1)
    colp1 = colf.reshape(NW, e_pad // (NW * ch1), ch1)
    rowp2 = rowf.reshape(NW, e_pad // (NW * ch2), ch2)
    colp2 = colf.reshape(NW, e_pad // (NW * ch2), ch2)

    ncp = _pad_up(nclass, 16)
    w2p = jnp.pad(W2, ((0, 0), (0, ncp - nclass)))

    hist = _sc_hist(rowp2, n_pad, e_pad // (NW * ch2))
    y1 = _tc_matmul(x, W1)
    dvec, z1 = _tc_scale(hist, y1)
    p1 = _sc_scatter(colp1, rowp1, z1, n_pad)
    z2 = _tc_mid(p1, z1, dvec, b1.reshape(1, -1), w2p)
    p2 = _sc_scatter(colp2, rowp2, z2, n_pad, sync_ring=True)
    out = _tc_out(p2, z2, dvec, b2.reshape(1, -1), nclass)
    return out
